# baseline (speedup 1.0000x reference)
import functools

import jax
import jax.numpy as jnp
from jax import lax
from jax.experimental import pallas as pl
from jax.experimental.pallas import tpu as pltpu
from jax.experimental.pallas import tpu_sc as plsc

N = 10000
IN = 256
HID = 256
HEADS = 8
NC = 64
E_RAW = 160000
E_REAL = E_RAW + N
EP = 170496
SH1 = EP // 16
SH2 = EP // 32
CW = 64
NCHUNK = HEADS * HID // CW
CPH = HID // CW


def _f32(*shape):
    return jax.ShapeDtypeStruct(shape, jnp.float32)


def _k1_body(x_ref, w_ref, v_ref, h_ref, asd_ref):
    c = pl.program_id(1)
    h_ref[...] = jnp.dot(x_ref[...], w_ref[0],
                         preferred_element_type=jnp.float32)

    @pl.when(c == 0)
    def _():
        asd_ref[...] = jnp.dot(x_ref[...], v_ref[...],
                               preferred_element_type=jnp.float32)


def _run_k1(x, W1, V1p):
    bn = 400
    grid = (N // bn, NCHUNK)
    return pl.pallas_call(
        _k1_body,
        grid=grid,
        in_specs=[
            pl.BlockSpec((bn, IN), lambda i, c: (i, 0)),
            pl.BlockSpec((1, IN, CW), lambda i, c: (c, 0, 0)),
            pl.BlockSpec((IN, 128), lambda i, c: (0, 0)),
        ],
        out_specs=[
            pl.BlockSpec((bn, CW), lambda i, c: (c * (N // bn) + i, 0)),
            pl.BlockSpec((bn, 128), lambda i, c: (i, 0)),
        ],
        out_shape=[_f32(NCHUNK * N, CW), _f32(N, 128)],
        compiler_params=pltpu.CompilerParams(
            dimension_semantics=("parallel", "arbitrary")),
    )(x, W1, V1p)


def _k2_body(o1_ref, b_ref, w_ref, out_ref):
    c = pl.program_id(1)
    v = o1_ref[0] + b_ref[0]
    h = jnp.where(v > 0, v, jnp.exp(jnp.minimum(v, 0.0)) - 1.0)
    contrib = jnp.dot(h, w_ref[0], preferred_element_type=jnp.float32)

    @pl.when(c == 0)
    def _():
        out_ref[...] = contrib

    @pl.when(c != 0)
    def _():
        out_ref[...] += contrib


def _run_k2(out1r, bias1cm, W2e):
    bn = 400
    grid = (N // bn, NCHUNK)
    return pl.pallas_call(
        _k2_body,
        grid=grid,
        in_specs=[
            pl.BlockSpec((1, bn, CW), lambda i, c: (c, i, 0)),
            pl.BlockSpec((1, 1, CW), lambda i, c: (c, 0, 0)),
            pl.BlockSpec((1, CW, 128), lambda i, c: (c, 0, 0)),
        ],
        out_specs=pl.BlockSpec((bn, 128), lambda i, c: (i, 0)),
        out_shape=_f32(N, 128),
        compiler_params=pltpu.CompilerParams(
            dimension_semantics=("parallel", "arbitrary")),
    )(out1r, bias1cm, W2e)


def _k3_body(p_ref, b_ref, out_ref):
    v = p_ref[0] + p_ref[1] + b_ref[...]
    m = jnp.max(v, axis=1, keepdims=True)
    ex = jnp.exp(v - m)
    lse = jnp.log(jnp.sum(ex, axis=1, keepdims=True))
    out_ref[...] = v - m - lse


def _run_k3(parts, bias2):
    bn = 400
    return pl.pallas_call(
        _k3_body,
        grid=(N // bn,),
        in_specs=[
            pl.BlockSpec((2, bn, NC), lambda i: (0, i, 0)),
            pl.BlockSpec((1, NC), lambda i: (0, 0)),
        ],
        out_specs=pl.BlockSpec((bn, NC), lambda i: (i, 0)),
        out_shape=_f32(N, NC),
    )(parts, bias2.reshape(1, NC))


_MESH = dict(core_axis_name="c", subcore_axis_name="s")
_SC_PARAMS = pltpu.CompilerParams(needs_layout_passes=False,
                                  use_tc_tiling_on_sc=False)


def _edge_p(aS16, aD16, b_vec, eid, valid_limit):
    e = aS16 + aD16
    e = jnp.where(e >= 0.0, e, 0.2 * e)
    p = jnp.exp(e - b_vec)
    return jnp.where(eid < valid_limit, p, 0.0)


def _bcast_lane(ref, lane):
    return plsc.load_gather(ref, [jnp.full((16,), lane, jnp.int32)])


def _alpha16(src_v, dst_v, aS_t, aD_t, s_tab, bh, base, off, iota):
    s16 = src_v[pl.ds(off, 16)]
    d16 = dst_v[pl.ds(off, 16)]
    aS16 = plsc.load_gather(aS_t, [s16])
    aD16 = plsc.load_gather(aD_t, [d16])
    p = _edge_p(aS16, aD16, bh, base + off + iota, E_REAL)
    sg = plsc.load_gather(s_tab, [d16 >> 4, d16 & 15])
    return s16, d16, p / (sg + 1e-16)


def _sweep(ng, width, rows_ref, rowbase, src_v, dst_v, aS_t, aD_t, s_tab, bh,
           base, iota, hbuf0, hbuf1, a_v, o_sh, sem0, sem1):
    def issue(g, buf, sem):
        offc = jnp.minimum(g * 16, (ng - 1) * 16)
        s16 = src_v[pl.ds(offc, 16)]
        pltpu.async_copy(rows_ref.at[rowbase + s16], buf, sem)

    def process(g, buf, sem):
        off = g * 16
        s16, d16, al = _alpha16(src_v, dst_v, aS_t, aD_t, s_tab, bh,
                                base, off, iota)
        a_v[...] = al
        pltpu.make_async_copy(rows_ref.at[rowbase + s16], buf, sem).wait()
        av = a_v[...]
        for r in range(16):
            ar = av[r]
            for cc in range(width // 16):
                buf[r, pl.ds(cc * 16, 16)] = buf[r, pl.ds(cc * 16, 16)] * ar
        pltpu.sync_copy(buf, o_sh.at[d16], add=True)

    issue(0, hbuf0, sem0)

    def pair(g2, _):
        a = 2 * g2
        issue(a + 1, hbuf1, sem1)
        process(a, hbuf0, sem0)
        issue(a + 2, hbuf0, sem0)
        process(a + 1, hbuf1, sem1)
        return 0

    lax.fori_loop(0, ng // 2, pair, 0)
    if ng % 2:
        process(ng - 1, hbuf0, sem0)
    else:
        pltpu.make_async_copy(rows_ref.at[rowbase + src_v[pl.ds(0, 16)]],
                              hbuf0, sem0).wait()


def _sweep_cached(ng, width, rows_ref, rowbase, src_v, dst_v, alf_v,
                  hbuf0, hbuf1, o_sh, sem0, sem1):
    def issue(g, buf, sem):
        offc = jnp.minimum(g * 16, (ng - 1) * 16)
        s16 = src_v[pl.ds(offc, 16)]
        pltpu.async_copy(rows_ref.at[rowbase + s16], buf, sem)

    def process(g, buf, sem):
        off = g * 16
        s16 = src_v[pl.ds(off, 16)]
        d16 = dst_v[pl.ds(off, 16)]
        av = alf_v[pl.ds(off, 16)]
        pltpu.make_async_copy(rows_ref.at[rowbase + s16], buf, sem).wait()
        for r in range(16):
            ar = av[r]
            for cc in range(width // 16):
                buf[r, pl.ds(cc * 16, 16)] = buf[r, pl.ds(cc * 16, 16)] * ar
        pltpu.sync_copy(buf, o_sh.at[d16], add=True)

    issue(0, hbuf0, sem0)

    def pair(g2, _):
        a = 2 * g2
        issue(a + 1, hbuf1, sem1)
        process(a, hbuf0, sem0)
        issue(a + 2, hbuf0, sem0)
        process(a + 1, hbuf1, sem1)
        return 0

    lax.fori_loop(0, ng // 2, pair, 0)
    if ng % 2:
        process(ng - 1, hbuf0, sem0)
    else:
        pltpu.make_async_copy(rows_ref.at[rowbase + src_v[pl.ds(0, 16)]],
                              hbuf0, sem0).wait()


def _stripe_out(o_sh, out_h, outbase, sid):
    pltpu.sync_copy(o_sh.at[pl.ds(sid * 624, 624)],
                    out_h.at[pl.ds(outbase + sid * 624, 624)])

    @pl.when(sid == 15)
    def _():
        pltpu.sync_copy(o_sh.at[pl.ds(9984, 16)],
                        out_h.at[pl.ds(outbase + 9984, 16)])


def _stripe_zero(z_h, o_sh, sid):
    pltpu.sync_copy(z_h.at[pl.ds(0, 624)], o_sh.at[pl.ds(sid * 624, 624)])

    @pl.when(sid == 15)
    def _():
        pltpu.sync_copy(z_h.at[pl.ds(0, 16)], o_sh.at[pl.ds(9984, 16)])


def _l1_sc(srcP, dstP, aS1T, aD1T, B1v, H1flat, idxall, z16, z64):
    mesh = plsc.VectorSubcoreMesh(**_MESH)

    @functools.partial(
        pl.kernel, mesh=mesh,
        out_type=_f32(NCHUNK * N, CW),
        compiler_params=_SC_PARAMS,
        scratch_types=[
            pltpu.VMEM((SH1,), jnp.int32),
            pltpu.VMEM((SH1,), jnp.int32),
            pltpu.VMEM((N,), jnp.float32),
            pltpu.VMEM((N,), jnp.float32),
            pltpu.VMEM((640, 16), jnp.float32),
            pltpu.VMEM((640, 16), jnp.float32),
            pltpu.VMEM((5, 128), jnp.int32),
            pltpu.VMEM((16, CW), jnp.float32),
            pltpu.VMEM((16, CW), jnp.float32),
            pltpu.VMEM((16,), jnp.float32),
            pltpu.VMEM((16,), jnp.float32),
            pltpu.VMEM((SH1,), jnp.float32),
            pltpu.VMEM_SHARED((2560, 16), jnp.float32),
            pltpu.VMEM_SHARED((N, CW), jnp.float32),
            pltpu.SemaphoreType.DMA,
            pltpu.SemaphoreType.DMA,
        ])
    def k(src_h, dst_h, aS_h, aD_h, b_h, h1_h, idx_h, z16_h, z64_h,
          out_h, src_v, dst_v, aS_t, aD_t, s_loc, s_tab, idx_v, hbuf0,
          hbuf1, a_v, b_v, alf_v, s_sh, o_sh, sem0, sem1):
        sid = lax.axis_index("s")
        cid = lax.axis_index("c")
        base = sid * SH1
        iota = lax.iota(jnp.int32, 16)

        pltpu.sync_copy(src_h.at[pl.ds(base, SH1)], src_v)
        pltpu.sync_copy(dst_h.at[pl.ds(base, SH1)], dst_v)

        pltpu.sync_copy(z16_h.at[pl.ds(0, 160)],
                        s_sh.at[pl.ds(sid * 160, 160)])
        plsc.subcore_barrier()

        for hh in range(HEADS // 2):
            h = cid * (HEADS // 2) + hh
            pltpu.sync_copy(aS_h.at[h], aS_t)
            pltpu.sync_copy(aD_h.at[h], aD_t)
            pltpu.sync_copy(z16_h, s_loc)
            pltpu.sync_copy(b_h.at[h], b_v)
            bh = b_v[...]

            def p1_body(g, _):
                off = g * 16
                s16 = src_v[pl.ds(off, 16)]
                d16 = dst_v[pl.ds(off, 16)]
                aS16 = plsc.load_gather(aS_t, [s16])
                aD16 = plsc.load_gather(aD_t, [d16])
                p = _edge_p(aS16, aD16, bh, base + off + iota, E_REAL)
                plsc.addupdate_scatter(s_loc, [d16 >> 4, d16 & 15], p)
                return 0

            lax.fori_loop(0, SH1 // 16, p1_body, 0)

            pltpu.sync_copy(idx_h.at[hh], idx_v)
            for turn in range(16):
                @pl.when(sid == turn)
                def _():
                    for j in range(5):
                        pltpu.sync_copy(s_loc.at[pl.ds(j * 128, 128)],
                                        s_sh.at[idx_v.at[j]], add=True)
                plsc.subcore_barrier()
        plsc.subcore_barrier()

        def head_body(hh, _):
            h = cid * (HEADS // 2) + hh
            pltpu.sync_copy(aS_h.at[h], aS_t)
            pltpu.sync_copy(aD_h.at[h], aD_t)
            pltpu.sync_copy(s_sh.at[pl.ds(hh * 640, 640)], s_tab)
            pltpu.sync_copy(b_h.at[h], b_v)
            bh = b_v[...]

            def acomp(g, _):
                off = g * 16
                _, _, al = _alpha16(src_v, dst_v, aS_t, aD_t, s_tab, bh,
                                    base, off, iota)
                alf_v[pl.ds(off, 16)] = al
                return 0

            lax.fori_loop(0, SH1 // 16, acomp, 0)

            def chunk_body(c4, _):
                kchunk = h * CPH + c4
                _stripe_zero(z64_h, o_sh, sid)
                plsc.subcore_barrier()
                rowbase = kchunk * N
                _sweep_cached(SH1 // 16, CW, h1_h, rowbase, src_v, dst_v,
                              alf_v, hbuf0, hbuf1, o_sh, sem0, sem1)
                plsc.subcore_barrier()
                _stripe_out(o_sh, out_h, rowbase, sid)
                plsc.subcore_barrier()
                return 0

            lax.fori_loop(0, CPH, chunk_body, 0)
            return 0

        lax.fori_loop(0, HEADS // 2, head_body, 0)

    return k(srcP, dstP, aS1T, aD1T, B1v, H1flat, idxall, z16, z64)


def _l2_sc(srcP, dstP, aS2T, aD2T, B2v, H2, idx640, z16, z64):
    mesh = plsc.VectorSubcoreMesh(**_MESH)

    @functools.partial(
        pl.kernel, mesh=mesh,
        out_type=_f32(2 * N, NC),
        compiler_params=_SC_PARAMS,
        scratch_types=[
            pltpu.VMEM((SH1,), jnp.int32),
            pltpu.VMEM((SH1,), jnp.int32),
            pltpu.VMEM((N,), jnp.float32),
            pltpu.VMEM((N,), jnp.float32),
            pltpu.VMEM((640, 16), jnp.float32),
            pltpu.VMEM((640, 16), jnp.float32),
            pltpu.VMEM((5, 128), jnp.int32),
            pltpu.VMEM((16, NC), jnp.float32),
            pltpu.VMEM((16, NC), jnp.float32),
            pltpu.VMEM((16,), jnp.float32),
            pltpu.VMEM((16,), jnp.float32),
            pltpu.VMEM_SHARED((640, 16), jnp.float32),
            pltpu.VMEM_SHARED((N, NC), jnp.float32),
            pltpu.SemaphoreType.DMA,
            pltpu.SemaphoreType.DMA,
        ])
    def k(src_h, dst_h, aS_h, aD_h, b_h, h2_h, idx_h, z16_h, z64_h,
          out_h, src_v, dst_v, aS_t, aD_t, s_loc, s_tab, idx_v, hbuf0,
          hbuf1, a_v, b_v, s_sh, o_sh, sem0, sem1):
        sid = lax.axis_index("s")
        cid = lax.axis_index("c")
        iota = lax.iota(jnp.int32, 16)

        pltpu.sync_copy(aS_h, aS_t)
        pltpu.sync_copy(aD_h, aD_t)
        pltpu.sync_copy(z16_h, s_loc)
        pltpu.sync_copy(idx_h, idx_v)
        pltpu.sync_copy(b_h.at[0], b_v)
        bh = b_v[...]

        pltpu.sync_copy(z16_h.at[pl.ds(0, 40)],
                        s_sh.at[pl.ds(sid * 40, 40)])
        _stripe_zero(z64_h, o_sh, sid)
        plsc.subcore_barrier()

        base1 = sid * SH1
        pltpu.sync_copy(src_h.at[pl.ds(base1, SH1)], src_v)
        pltpu.sync_copy(dst_h.at[pl.ds(base1, SH1)], dst_v)

        def p1_body(g, _):
            off = g * 16
            s16 = src_v[pl.ds(off, 16)]
            d16 = dst_v[pl.ds(off, 16)]
            aS16 = plsc.load_gather(aS_t, [s16])
            aD16 = plsc.load_gather(aD_t, [d16])
            p = _edge_p(aS16, aD16, bh, base1 + off + iota, E_REAL)
            plsc.addupdate_scatter(s_loc, [d16 >> 4, d16 & 15], p)
            return 0

        lax.fori_loop(0, SH1 // 16, p1_body, 0)
        for turn in range(16):
            @pl.when(sid == turn)
            def _():
                for j in range(5):
                    pltpu.sync_copy(s_loc.at[pl.ds(j * 128, 128)],
                                    s_sh.at[idx_v.at[j]], add=True)
            plsc.subcore_barrier()
        plsc.subcore_barrier()
        pltpu.sync_copy(s_sh, s_tab)

        base2 = (cid * 16 + sid) * SH2
        pltpu.sync_copy(src_h.at[pl.ds(base2, SH2)],
                        src_v.at[pl.ds(0, SH2)])
        pltpu.sync_copy(dst_h.at[pl.ds(base2, SH2)],
                        dst_v.at[pl.ds(0, SH2)])
        _sweep(SH2 // 16, NC, h2_h, 0, src_v, dst_v, aS_t, aD_t, s_tab, bh,
               base2, iota, hbuf0, hbuf1, a_v, o_sh, sem0, sem1)
        plsc.subcore_barrier()
        _stripe_out(o_sh, out_h, cid * N, sid)

    return k(srcP, dstP, aS2T, aD2T, B2v, H2, idx640, z16, z64)


@jax.jit
def kernel(x, edge_index, W1, att_src1, att_dst1, bias1,
           W2, att_src2, att_dst2, bias2):
    npad = EP - E_REAL
    loop = jnp.arange(N, dtype=jnp.int32)
    padi = (jnp.arange(npad, dtype=jnp.int32) * 17) % N
    srcP = jnp.concatenate([edge_index[0].astype(jnp.int32), loop, padi])
    dstP = jnp.concatenate([edge_index[1].astype(jnp.int32), loop, padi])

    W1r = W1.reshape(IN, HEADS, HID)
    V1s = jnp.einsum('ihc,hc->ih', W1r, att_src1)
    V1d = jnp.einsum('ihc,hc->ih', W1r, att_dst1)
    V1p = jnp.zeros((IN, 128), jnp.float32)
    V1p = V1p.at[:, :HEADS].set(V1s).at[:, HEADS:2 * HEADS].set(V1d)

    W1cm = W1.reshape(IN, NCHUNK, CW).transpose(1, 0, 2)
    H1flat, asd = _run_k1(x, W1cm, V1p)
    aS1 = asd[:, :HEADS]
    aD1 = asd[:, HEADS:2 * HEADS]
    aS1T = aS1.T.reshape(HEADS, N)
    aD1T = aD1.T.reshape(HEADS, N)
    B1 = jnp.max(aS1, axis=0) + jnp.max(aD1, axis=0)
    B1v = jnp.broadcast_to(B1.reshape(HEADS, 1), (HEADS, 16))

    idxall = jnp.arange((HEADS // 2) * 640,
                        dtype=jnp.int32).reshape(HEADS // 2, 5, 128)
    z16 = jnp.zeros((640, 16), jnp.float32)
    zL1 = jnp.zeros((625, CW), jnp.float32)
    zL2 = jnp.zeros((625, NC), jnp.float32)

    out1flat = _l1_sc(srcP, dstP, aS1T, aD1T, B1v, H1flat, idxall, z16, zL1)

    out1r = out1flat.reshape(NCHUNK, N, CW)
    bias1cm = bias1.reshape(NCHUNK, 1, CW)
    W2e = jnp.zeros((NCHUNK, CW, 128), jnp.float32)
    W2r = W2.reshape(NCHUNK, CW, NC)
    V2s = (W2 @ att_src2[0]).reshape(NCHUNK, CW)
    V2d = (W2 @ att_dst2[0]).reshape(NCHUNK, CW)
    W2e = W2e.at[:, :, :NC].set(W2r)
    W2e = W2e.at[:, :, NC].set(V2s).at[:, :, NC + 1].set(V2d)

    H2e = _run_k2(out1r, bias1cm, W2e)
    H2 = H2e[:, :NC]
    aS2 = H2e[:, NC]
    aD2 = H2e[:, NC + 1]
    B2 = jnp.max(aS2) + jnp.max(aD2)
    B2v = jnp.full((1, 16), B2, jnp.float32)
    idx640 = jnp.arange(640, dtype=jnp.int32).reshape(5, 128)

    parts = _l2_sc(srcP, dstP, aS2, aD2, B2v, H2, idx640, z16, zL2)

    return _run_k3(parts.reshape(2, N, NC), bias2)


# speedup vs baseline: 1.3656x; 1.3656x over previous
import functools

import jax
import jax.numpy as jnp
from jax import lax
from jax.experimental import pallas as pl
from jax.experimental.pallas import tpu as pltpu
from jax.experimental.pallas import tpu_sc as plsc

N = 10000
IN = 256
HID = 256
HEADS = 8
NC = 64
E_RAW = 160000
E_REAL = E_RAW + N
EP = 170496
SH1 = EP // 16
SH2 = EP // 32
CW = 64
NCHUNK = HEADS * HID // CW
CPH = HID // CW


def _f32(*shape):
    return jax.ShapeDtypeStruct(shape, jnp.float32)


def _k1_body(x_ref, w_ref, v_ref, h_ref, asd_ref):
    c = pl.program_id(1)
    h_ref[...] = jnp.dot(x_ref[...], w_ref[0],
                         preferred_element_type=jnp.float32)

    @pl.when(c == 0)
    def _():
        asd_ref[...] = jnp.dot(x_ref[...], v_ref[...],
                               preferred_element_type=jnp.float32)


def _run_k1(x, W1, V1p):
    bn = 400
    grid = (N // bn, NCHUNK)
    return pl.pallas_call(
        _k1_body,
        grid=grid,
        in_specs=[
            pl.BlockSpec((bn, IN), lambda i, c: (i, 0)),
            pl.BlockSpec((1, IN, CW), lambda i, c: (c, 0, 0)),
            pl.BlockSpec((IN, 128), lambda i, c: (0, 0)),
        ],
        out_specs=[
            pl.BlockSpec((bn, CW), lambda i, c: (c * (N // bn) + i, 0)),
            pl.BlockSpec((bn, 128), lambda i, c: (i, 0)),
        ],
        out_shape=[_f32(NCHUNK * N, CW), _f32(N, 128)],
        compiler_params=pltpu.CompilerParams(
            dimension_semantics=("parallel", "arbitrary")),
    )(x, W1, V1p)


def _k2_body(o1_ref, b_ref, w_ref, out_ref):
    c = pl.program_id(1)
    v = o1_ref[0] + b_ref[0]
    h = jnp.where(v > 0, v, jnp.exp(jnp.minimum(v, 0.0)) - 1.0)
    contrib = jnp.dot(h, w_ref[0], preferred_element_type=jnp.float32)

    @pl.when(c == 0)
    def _():
        out_ref[...] = contrib

    @pl.when(c != 0)
    def _():
        out_ref[...] += contrib


def _run_k2(out1r, bias1cm, W2e):
    bn = 400
    grid = (N // bn, NCHUNK)
    return pl.pallas_call(
        _k2_body,
        grid=grid,
        in_specs=[
            pl.BlockSpec((1, bn, CW), lambda i, c: (c, i, 0)),
            pl.BlockSpec((1, 1, CW), lambda i, c: (c, 0, 0)),
            pl.BlockSpec((1, CW, 128), lambda i, c: (c, 0, 0)),
        ],
        out_specs=pl.BlockSpec((bn, 128), lambda i, c: (i, 0)),
        out_shape=_f32(N, 128),
        compiler_params=pltpu.CompilerParams(
            dimension_semantics=("parallel", "arbitrary")),
    )(out1r, bias1cm, W2e)


def _k3_body(p_ref, b_ref, out_ref):
    v = p_ref[0] + p_ref[1] + b_ref[...]
    m = jnp.max(v, axis=1, keepdims=True)
    ex = jnp.exp(v - m)
    lse = jnp.log(jnp.sum(ex, axis=1, keepdims=True))
    out_ref[...] = v - m - lse


def _run_k3(parts, bias2):
    bn = 400
    return pl.pallas_call(
        _k3_body,
        grid=(N // bn,),
        in_specs=[
            pl.BlockSpec((2, bn, NC), lambda i: (0, i, 0)),
            pl.BlockSpec((1, NC), lambda i: (0, 0)),
        ],
        out_specs=pl.BlockSpec((bn, NC), lambda i: (i, 0)),
        out_shape=_f32(N, NC),
    )(parts, bias2.reshape(1, NC))


_MESH = dict(core_axis_name="c", subcore_axis_name="s")
_SC_PARAMS = pltpu.CompilerParams(needs_layout_passes=False,
                                  use_tc_tiling_on_sc=False)


def _edge_p(aS16, aD16, b_vec, eid, valid_limit):
    e = aS16 + aD16
    e = jnp.where(e >= 0.0, e, 0.2 * e)
    p = jnp.exp(e - b_vec)
    return jnp.where(eid < valid_limit, p, 0.0)


def _bcast_lane(ref, lane):
    return plsc.load_gather(ref, [jnp.full((16,), lane, jnp.int32)])


def _alpha16(src_v, dst_v, aS_t, aD_t, s_tab, bh, base, off, iota):
    s16 = src_v[pl.ds(off, 16)]
    d16 = dst_v[pl.ds(off, 16)]
    aS16 = plsc.load_gather(aS_t, [s16])
    aD16 = plsc.load_gather(aD_t, [d16])
    p = _edge_p(aS16, aD16, bh, base + off + iota, E_REAL)
    sg = plsc.load_gather(s_tab, [d16 >> 4, d16 & 15])
    return s16, d16, p / (sg + 1e-16)


def _sweep(ng, width, rows_ref, rowbase, src_v, dst_v, aS_t, aD_t, s_tab, bh,
           base, iota, hbuf0, hbuf1, a_v, o_sh, sem0, sem1):
    def issue(g, buf, sem):
        offc = jnp.minimum(g * 16, (ng - 1) * 16)
        s16 = src_v[pl.ds(offc, 16)]
        pltpu.async_copy(rows_ref.at[rowbase + s16], buf, sem)

    def process(g, buf, sem):
        off = g * 16
        s16, d16, al = _alpha16(src_v, dst_v, aS_t, aD_t, s_tab, bh,
                                base, off, iota)
        a_v[...] = al
        pltpu.make_async_copy(rows_ref.at[rowbase + s16], buf, sem).wait()
        av = a_v[...]
        for r in range(16):
            ar = av[r]
            for cc in range(width // 16):
                buf[r, pl.ds(cc * 16, 16)] = buf[r, pl.ds(cc * 16, 16)] * ar
        pltpu.sync_copy(buf, o_sh.at[d16], add=True)

    issue(0, hbuf0, sem0)

    def pair(g2, _):
        a = 2 * g2
        issue(a + 1, hbuf1, sem1)
        process(a, hbuf0, sem0)
        issue(a + 2, hbuf0, sem0)
        process(a + 1, hbuf1, sem1)
        return 0

    lax.fori_loop(0, ng // 2, pair, 0)
    if ng % 2:
        process(ng - 1, hbuf0, sem0)
    else:
        pltpu.make_async_copy(rows_ref.at[rowbase + src_v[pl.ds(0, 16)]],
                              hbuf0, sem0).wait()


def _sweep_async(ng, width, rows_ref, rowbase, src_v, dst_v, alf_v,
                 bufs, gsems, ssems, o_sh):
    assert ng % 4 == 2 and ng >= 6

    def issue(g, b):
        s16 = src_v[pl.ds(g * 16, 16)]
        pltpu.async_copy(rows_ref.at[rowbase + s16], bufs[b], gsems[b])

    def wait_scatter(b):
        pltpu.make_async_copy(bufs[b], o_sh.at[dst_v[pl.ds(0, 16)]],
                              ssems[b]).wait()

    def process(g, b):
        off = g * 16
        s16 = src_v[pl.ds(off, 16)]
        d16 = dst_v[pl.ds(off, 16)]
        av = alf_v[pl.ds(off, 16)]
        buf = bufs[b]
        pltpu.make_async_copy(rows_ref.at[rowbase + s16], buf,
                              gsems[b]).wait()
        for r in range(16):
            ar = av[r]
            for cc in range(width // 16):
                buf[r, pl.ds(cc * 16, 16)] = buf[r, pl.ds(cc * 16, 16)] * ar
        pltpu.async_copy(buf, o_sh.at[d16], ssems[b], add=True)

    issue(0, 0)
    issue(1, 1)

    def quad(q, _):
        g0 = q * 4
        for b in range(4):
            g = g0 + b
            bn = (b + 2) % 4

            @pl.when(g >= 2)
            def _(bn=bn):
                wait_scatter(bn)

            issue(g + 2, bn)
            process(g, b)
        return 0

    lax.fori_loop(0, ng // 4, quad, 0)
    for g in (ng - 2, ng - 1):
        b = g % 4
        wait_scatter((b + 2) % 4)
        process(g, b)
    for b in (0, 1):
        wait_scatter(b)


def _stripe_out(o_sh, out_h, outbase, sid):
    pltpu.sync_copy(o_sh.at[pl.ds(sid * 624, 624)],
                    out_h.at[pl.ds(outbase + sid * 624, 624)])

    @pl.when(sid == 15)
    def _():
        pltpu.sync_copy(o_sh.at[pl.ds(9984, 16)],
                        out_h.at[pl.ds(outbase + 9984, 16)])


def _stripe_zero(z_h, o_sh, sid):
    pltpu.sync_copy(z_h.at[pl.ds(0, 624)], o_sh.at[pl.ds(sid * 624, 624)])

    @pl.when(sid == 15)
    def _():
        pltpu.sync_copy(z_h.at[pl.ds(0, 16)], o_sh.at[pl.ds(9984, 16)])


def _l1_sc(srcP, dstP, aS1T, aD1T, B1v, H1flat, idxall, z16, z64):
    mesh = plsc.VectorSubcoreMesh(**_MESH)

    @functools.partial(
        pl.kernel, mesh=mesh,
        out_type=_f32(NCHUNK * N, CW),
        compiler_params=_SC_PARAMS,
        scratch_types=[
            pltpu.VMEM((SH1,), jnp.int32),
            pltpu.VMEM((SH1,), jnp.int32),
            pltpu.VMEM((N,), jnp.float32),
            pltpu.VMEM((N,), jnp.float32),
            pltpu.VMEM((640, 16), jnp.float32),
            pltpu.VMEM((640, 16), jnp.float32),
            pltpu.VMEM((5, 128), jnp.int32),
            pltpu.VMEM((16, CW), jnp.float32),
            pltpu.VMEM((16, CW), jnp.float32),
            pltpu.VMEM((16, CW), jnp.float32),
            pltpu.VMEM((16, CW), jnp.float32),
            pltpu.VMEM((16,), jnp.float32),
            pltpu.VMEM((SH1,), jnp.float32),
            pltpu.VMEM_SHARED((2560, 16), jnp.float32),
            pltpu.VMEM_SHARED((N, CW), jnp.float32),
            pltpu.SemaphoreType.DMA,
            pltpu.SemaphoreType.DMA,
            pltpu.SemaphoreType.DMA,
            pltpu.SemaphoreType.DMA,
            pltpu.SemaphoreType.DMA,
            pltpu.SemaphoreType.DMA,
            pltpu.SemaphoreType.DMA,
            pltpu.SemaphoreType.DMA,
        ])
    def k(src_h, dst_h, aS_h, aD_h, b_h, h1_h, idx_h, z16_h, z64_h,
          out_h, src_v, dst_v, aS_t, aD_t, s_loc, s_tab, idx_v, hbuf0,
          hbuf1, hbuf2, hbuf3, b_v, alf_v, s_sh, o_sh,
          gs0, gs1, gs2, gs3, ss0, ss1, ss2, ss3):
        sid = lax.axis_index("s")
        cid = lax.axis_index("c")
        base = sid * SH1
        iota = lax.iota(jnp.int32, 16)

        pltpu.sync_copy(src_h.at[pl.ds(base, SH1)], src_v)
        pltpu.sync_copy(dst_h.at[pl.ds(base, SH1)], dst_v)

        pltpu.sync_copy(z16_h.at[pl.ds(0, 160)],
                        s_sh.at[pl.ds(sid * 160, 160)])
        plsc.subcore_barrier()

        for hh in range(HEADS // 2):
            h = cid * (HEADS // 2) + hh
            pltpu.sync_copy(aS_h.at[h], aS_t)
            pltpu.sync_copy(aD_h.at[h], aD_t)
            pltpu.sync_copy(z16_h, s_loc)
            pltpu.sync_copy(b_h.at[h], b_v)
            bh = b_v[...]

            def p1_body(g, _):
                off = g * 16
                s16 = src_v[pl.ds(off, 16)]
                d16 = dst_v[pl.ds(off, 16)]
                aS16 = plsc.load_gather(aS_t, [s16])
                aD16 = plsc.load_gather(aD_t, [d16])
                p = _edge_p(aS16, aD16, bh, base + off + iota, E_REAL)
                plsc.addupdate_scatter(s_loc, [d16 >> 4, d16 & 15], p)
                return 0

            lax.fori_loop(0, SH1 // 16, p1_body, 0)

            pltpu.sync_copy(idx_h.at[hh], idx_v)
            for turn in range(16):
                @pl.when(sid == turn)
                def _():
                    for j in range(5):
                        pltpu.sync_copy(s_loc.at[pl.ds(j * 128, 128)],
                                        s_sh.at[idx_v.at[j]], add=True)
                plsc.subcore_barrier()
        plsc.subcore_barrier()

        def head_body(hh, _):
            h = cid * (HEADS // 2) + hh
            pltpu.sync_copy(aS_h.at[h], aS_t)
            pltpu.sync_copy(aD_h.at[h], aD_t)
            pltpu.sync_copy(s_sh.at[pl.ds(hh * 640, 640)], s_tab)
            pltpu.sync_copy(b_h.at[h], b_v)
            bh = b_v[...]

            def acomp(g, _):
                off = g * 16
                _, _, al = _alpha16(src_v, dst_v, aS_t, aD_t, s_tab, bh,
                                    base, off, iota)
                alf_v[pl.ds(off, 16)] = al
                return 0

            lax.fori_loop(0, SH1 // 16, acomp, 0)

            def chunk_body(c4, _):
                kchunk = h * CPH + c4
                _stripe_zero(z64_h, o_sh, sid)
                plsc.subcore_barrier()
                rowbase = kchunk * N
                _sweep_async(SH1 // 16, CW, h1_h, rowbase, src_v, dst_v,
                             alf_v, (hbuf0, hbuf1, hbuf2, hbuf3),
                             (gs0, gs1, gs2, gs3), (ss0, ss1, ss2, ss3),
                             o_sh)
                plsc.subcore_barrier()
                _stripe_out(o_sh, out_h, rowbase, sid)
                plsc.subcore_barrier()
                return 0

            lax.fori_loop(0, CPH, chunk_body, 0)
            return 0

        lax.fori_loop(0, HEADS // 2, head_body, 0)

    return k(srcP, dstP, aS1T, aD1T, B1v, H1flat, idxall, z16, z64)


def _l2_sc(srcP, dstP, aS2T, aD2T, B2v, H2, idx640, z16, z64):
    mesh = plsc.VectorSubcoreMesh(**_MESH)

    @functools.partial(
        pl.kernel, mesh=mesh,
        out_type=_f32(2 * N, NC),
        compiler_params=_SC_PARAMS,
        scratch_types=[
            pltpu.VMEM((SH1,), jnp.int32),
            pltpu.VMEM((SH1,), jnp.int32),
            pltpu.VMEM((N,), jnp.float32),
            pltpu.VMEM((N,), jnp.float32),
            pltpu.VMEM((640, 16), jnp.float32),
            pltpu.VMEM((640, 16), jnp.float32),
            pltpu.VMEM((5, 128), jnp.int32),
            pltpu.VMEM((16, NC), jnp.float32),
            pltpu.VMEM((16, NC), jnp.float32),
            pltpu.VMEM((16,), jnp.float32),
            pltpu.VMEM((16,), jnp.float32),
            pltpu.VMEM_SHARED((640, 16), jnp.float32),
            pltpu.VMEM_SHARED((N, NC), jnp.float32),
            pltpu.SemaphoreType.DMA,
            pltpu.SemaphoreType.DMA,
        ])
    def k(src_h, dst_h, aS_h, aD_h, b_h, h2_h, idx_h, z16_h, z64_h,
          out_h, src_v, dst_v, aS_t, aD_t, s_loc, s_tab, idx_v, hbuf0,
          hbuf1, a_v, b_v, s_sh, o_sh, sem0, sem1):
        sid = lax.axis_index("s")
        cid = lax.axis_index("c")
        iota = lax.iota(jnp.int32, 16)

        pltpu.sync_copy(aS_h, aS_t)
        pltpu.sync_copy(aD_h, aD_t)
        pltpu.sync_copy(z16_h, s_loc)
        pltpu.sync_copy(idx_h, idx_v)
        pltpu.sync_copy(b_h.at[0], b_v)
        bh = b_v[...]

        pltpu.sync_copy(z16_h.at[pl.ds(0, 40)],
                        s_sh.at[pl.ds(sid * 40, 40)])
        _stripe_zero(z64_h, o_sh, sid)
        plsc.subcore_barrier()

        base1 = sid * SH1
        pltpu.sync_copy(src_h.at[pl.ds(base1, SH1)], src_v)
        pltpu.sync_copy(dst_h.at[pl.ds(base1, SH1)], dst_v)

        def p1_body(g, _):
            off = g * 16
            s16 = src_v[pl.ds(off, 16)]
            d16 = dst_v[pl.ds(off, 16)]
            aS16 = plsc.load_gather(aS_t, [s16])
            aD16 = plsc.load_gather(aD_t, [d16])
            p = _edge_p(aS16, aD16, bh, base1 + off + iota, E_REAL)
            plsc.addupdate_scatter(s_loc, [d16 >> 4, d16 & 15], p)
            return 0

        lax.fori_loop(0, SH1 // 16, p1_body, 0)
        for turn in range(16):
            @pl.when(sid == turn)
            def _():
                for j in range(5):
                    pltpu.sync_copy(s_loc.at[pl.ds(j * 128, 128)],
                                    s_sh.at[idx_v.at[j]], add=True)
            plsc.subcore_barrier()
        plsc.subcore_barrier()
        pltpu.sync_copy(s_sh, s_tab)

        base2 = (cid * 16 + sid) * SH2
        pltpu.sync_copy(src_h.at[pl.ds(base2, SH2)],
                        src_v.at[pl.ds(0, SH2)])
        pltpu.sync_copy(dst_h.at[pl.ds(base2, SH2)],
                        dst_v.at[pl.ds(0, SH2)])
        _sweep(SH2 // 16, NC, h2_h, 0, src_v, dst_v, aS_t, aD_t, s_tab, bh,
               base2, iota, hbuf0, hbuf1, a_v, o_sh, sem0, sem1)
        plsc.subcore_barrier()
        _stripe_out(o_sh, out_h, cid * N, sid)

    return k(srcP, dstP, aS2T, aD2T, B2v, H2, idx640, z16, z64)


@jax.jit
def kernel(x, edge_index, W1, att_src1, att_dst1, bias1,
           W2, att_src2, att_dst2, bias2):
    npad = EP - E_REAL
    loop = jnp.arange(N, dtype=jnp.int32)
    padi = (jnp.arange(npad, dtype=jnp.int32) * 17) % N
    srcP = jnp.concatenate([edge_index[0].astype(jnp.int32), loop, padi])
    dstP = jnp.concatenate([edge_index[1].astype(jnp.int32), loop, padi])

    W1r = W1.reshape(IN, HEADS, HID)
    V1s = jnp.einsum('ihc,hc->ih', W1r, att_src1)
    V1d = jnp.einsum('ihc,hc->ih', W1r, att_dst1)
    V1p = jnp.zeros((IN, 128), jnp.float32)
    V1p = V1p.at[:, :HEADS].set(V1s).at[:, HEADS:2 * HEADS].set(V1d)

    W1cm = W1.reshape(IN, NCHUNK, CW).transpose(1, 0, 2)
    H1flat, asd = _run_k1(x, W1cm, V1p)
    aS1 = asd[:, :HEADS]
    aD1 = asd[:, HEADS:2 * HEADS]
    aS1T = aS1.T.reshape(HEADS, N)
    aD1T = aD1.T.reshape(HEADS, N)
    B1 = jnp.max(aS1, axis=0) + jnp.max(aD1, axis=0)
    B1v = jnp.broadcast_to(B1.reshape(HEADS, 1), (HEADS, 16))

    idxall = jnp.arange((HEADS // 2) * 640,
                        dtype=jnp.int32).reshape(HEADS // 2, 5, 128)
    z16 = jnp.zeros((640, 16), jnp.float32)
    zL1 = jnp.zeros((625, CW), jnp.float32)
    zL2 = jnp.zeros((625, NC), jnp.float32)

    out1flat = _l1_sc(srcP, dstP, aS1T, aD1T, B1v, H1flat, idxall, z16, zL1)

    out1r = out1flat.reshape(NCHUNK, N, CW)
    bias1cm = bias1.reshape(NCHUNK, 1, CW)
    W2e = jnp.zeros((NCHUNK, CW, 128), jnp.float32)
    W2r = W2.reshape(NCHUNK, CW, NC)
    V2s = (W2 @ att_src2[0]).reshape(NCHUNK, CW)
    V2d = (W2 @ att_dst2[0]).reshape(NCHUNK, CW)
    W2e = W2e.at[:, :, :NC].set(W2r)
    W2e = W2e.at[:, :, NC].set(V2s).at[:, :, NC + 1].set(V2d)

    H2e = _run_k2(out1r, bias1cm, W2e)
    H2 = H2e[:, :NC]
    aS2 = H2e[:, NC]
    aD2 = H2e[:, NC + 1]
    B2 = jnp.max(aS2) + jnp.max(aD2)
    B2v = jnp.full((1, 16), B2, jnp.float32)
    idx640 = jnp.arange(640, dtype=jnp.int32).reshape(5, 128)

    parts = _l2_sc(srcP, dstP, aS2, aD2, B2v, H2, idx640, z16, zL2)

    return _run_k3(parts.reshape(2, N, NC), bias2)
